# baseline (speedup 1.0000x reference)
import math

import jax
import jax.numpy as jnp
import numpy as np
from jax import lax
from jax.experimental import pallas as pl
from jax.experimental.pallas import tpu as pltpu
from jax.experimental.pallas import tpu_sc as plsc

D_MODEL = 128
SEQ = 50
BATCH = 4096
NUM_WORKERS = 32
BPW = BATCH // NUM_WORKERS
NVEC = D_MODEL // 16


def _pe_table(seq_len: int, d_model: int) -> np.ndarray:
    position = np.arange(seq_len, dtype=np.float32)[:, None]
    div_term = np.exp(
        np.arange(0, d_model, 2, dtype=np.float32) * (-math.log(10000.0) / d_model)
    )
    pe = np.zeros((seq_len, d_model), dtype=np.float32)
    pe[:, 0::2] = np.sin(position * div_term)
    pe[:, 1::2] = np.cos(position * div_term)
    return pe


_PE = _pe_table(SEQ, D_MODEL)

_MESH = plsc.VectorSubcoreMesh(core_axis_name="c", subcore_axis_name="s")

NGB = 4
NSB = 2

_SCRATCH = [
    pltpu.VMEM((SEQ, D_MODEL), jnp.float32),
    pltpu.VMEM((SEQ, BPW), jnp.int32),
    *[pltpu.VMEM((BPW, D_MODEL), jnp.float32) for _ in range(NGB)],
    *[pltpu.VMEM((BPW, D_MODEL), jnp.float32) for _ in range(NSB)],
    *[pltpu.SemaphoreType.DMA for _ in range(NGB + NSB)],
]


def _sc_body(xT_hbm, table_hbm, pe_hbm, out_hbm,
                 pe_v, idx_v, g0, g1, g2, g3, s0, s1,
                 sem_g0, sem_g1, sem_g2, sem_g3, sem_s0, sem_s1):
    wid = lax.axis_index("s") * 2 + lax.axis_index("c")
    b0 = wid * BPW

    gbuf = (g0, g1, g2, g3)
    sbuf = (s0, s1)
    sem_g = (sem_g0, sem_g1, sem_g2, sem_g3)
    sem_s = (sem_s0, sem_s1)

    pltpu.sync_copy(xT_hbm.at[:, pl.ds(b0, BPW)], idx_v)
    pe_copy = pltpu.async_copy(pe_hbm, pe_v, sem_s0)

    def start_gather(l, b):
        pltpu.async_copy(table_hbm.at[idx_v.at[l]], gbuf[b], sem_g[b])

    def wait_gather(l, b):
        pltpu.make_async_copy(table_hbm.at[idx_v.at[l]], gbuf[b], sem_g[b]).wait()

    def start_scatter(l, b):
        pltpu.async_copy(sbuf[b], out_hbm.at[l, pl.ds(b0, BPW)], sem_s[b])

    def wait_scatter(l, b):
        pltpu.make_async_copy(sbuf[b], out_hbm.at[l, pl.ds(b0, BPW)], sem_s[b]).wait()

    def compute(l, b, bs):
        gb, sb = gbuf[b], sbuf[bs]
        pe_row = [pe_v[l, pl.ds(c * 16, 16)] for c in range(NVEC)]

        @plsc.parallel_loop(0, BPW, unroll=4)
        def _(r):
            for c in range(NVEC):
                sl = pl.ds(c * 16, 16)
                sb[r, sl] = gb[r, sl] + pe_row[c]

    def step(l, bg, bs, first, last):
        wait_gather(l, bg)
        if not first:
            wait_scatter(l - NSB, bs)
        compute(l, bg, bs)
        start_scatter(l, bs)
        if not last:
            start_gather(l + NGB, bg)

    for b in range(NGB):
        start_gather(b, b)
    pe_copy.wait()

    for l in range(NSB):
        step(l, l % NGB, l % NSB, True, False)

    @pl.loop(NSB, SEQ - NGB, step=NGB)
    def _(l):
        for j in range(NGB):
            step(l + j, (NSB + j) % NGB, j % NSB, False, False)

    for l in range(SEQ - NGB, SEQ):
        step(l, l % NGB, l % NSB, False, True)

    for l in range(SEQ - NSB, SEQ):
        wait_scatter(l, l % NSB)


_sc_embed_pe = pl.kernel(
    _sc_body,
    out_type=jax.ShapeDtypeStruct((SEQ, BATCH, D_MODEL), jnp.float32),
    mesh=_MESH,
    scratch_types=_SCRATCH,
)


def kernel(x, table):
    pe = jnp.asarray(_PE)
    xT = x.T
    return _sc_embed_pe(xT, table, pe).transpose(1, 0, 2)


# speedup vs baseline: 1.0012x; 1.0012x over previous
import math

import jax
import jax.numpy as jnp
import numpy as np
from jax import lax
from jax.experimental import pallas as pl
from jax.experimental.pallas import tpu as pltpu
from jax.experimental.pallas import tpu_sc as plsc

D_MODEL = 128
SEQ = 50
BATCH = 4096
NUM_WORKERS = 32
BPW = BATCH // NUM_WORKERS
NVEC = D_MODEL // 16


def _pe_table(seq_len: int, d_model: int) -> np.ndarray:
    position = np.arange(seq_len, dtype=np.float32)[:, None]
    div_term = np.exp(
        np.arange(0, d_model, 2, dtype=np.float32) * (-math.log(10000.0) / d_model)
    )
    pe = np.zeros((seq_len, d_model), dtype=np.float32)
    pe[:, 0::2] = np.sin(position * div_term)
    pe[:, 1::2] = np.cos(position * div_term)
    return pe


_PE = _pe_table(SEQ, D_MODEL)

_MESH = plsc.VectorSubcoreMesh(core_axis_name="c", subcore_axis_name="s")

NGB = 4
NSB = 2

_SCRATCH = [
    pltpu.VMEM((SEQ, D_MODEL), jnp.float32),
    pltpu.VMEM((SEQ, BPW), jnp.int32),
    *[pltpu.VMEM((BPW, D_MODEL), jnp.float32) for _ in range(NGB)],
    *[pltpu.VMEM((BPW, D_MODEL), jnp.float32) for _ in range(NSB)],
    *[pltpu.SemaphoreType.DMA for _ in range(NGB + NSB)],
]


def _sc_body(xT_hbm, table_hbm, pe_hbm, out_hbm,
                 pe_v, idx_v, g0, g1, g2, g3, s0, s1,
                 sem_g0, sem_g1, sem_g2, sem_g3, sem_s0, sem_s1):
    wid = lax.axis_index("s") * 2 + lax.axis_index("c")
    b0 = wid * BPW

    gbuf = (g0, g1, g2, g3)
    sbuf = (s0, s1)
    sem_g = (sem_g0, sem_g1, sem_g2, sem_g3)
    sem_s = (sem_s0, sem_s1)

    IDX_HEAD = 8
    pltpu.sync_copy(xT_hbm.at[pl.ds(0, IDX_HEAD), pl.ds(b0, BPW)],
                    idx_v.at[pl.ds(0, IDX_HEAD)])
    pe_copy = pltpu.async_copy(pe_hbm, pe_v, sem_s0)
    idx_rest = pltpu.async_copy(
        xT_hbm.at[pl.ds(IDX_HEAD, SEQ - IDX_HEAD), pl.ds(b0, BPW)],
        idx_v.at[pl.ds(IDX_HEAD, SEQ - IDX_HEAD)], sem_s1)

    def start_gather(l, b):
        pltpu.async_copy(table_hbm.at[idx_v.at[l]], gbuf[b], sem_g[b])

    def wait_gather(l, b):
        pltpu.make_async_copy(table_hbm.at[idx_v.at[l]], gbuf[b], sem_g[b]).wait()

    def start_scatter(l, b):
        pltpu.async_copy(sbuf[b], out_hbm.at[l, pl.ds(b0, BPW)], sem_s[b])

    def wait_scatter(l, b):
        pltpu.make_async_copy(sbuf[b], out_hbm.at[l, pl.ds(b0, BPW)], sem_s[b]).wait()

    def compute(l, b, bs):
        gb, sb = gbuf[b], sbuf[bs]
        pe_row = [pe_v[l, pl.ds(c * 16, 16)] for c in range(NVEC)]

        @plsc.parallel_loop(0, BPW, unroll=4)
        def _(r):
            for c in range(NVEC):
                sl = pl.ds(c * 16, 16)
                sb[r, sl] = gb[r, sl] + pe_row[c]

    def step(l, bg, bs, first, last):
        wait_gather(l, bg)
        if not first:
            wait_scatter(l - NSB, bs)
        compute(l, bg, bs)
        start_scatter(l, bs)
        if not last:
            start_gather(l + NGB, bg)

    for b in range(NGB):
        start_gather(b, b)
    pe_copy.wait()
    idx_rest.wait()

    for l in range(NSB):
        step(l, l % NGB, l % NSB, True, False)

    @pl.loop(NSB, SEQ - NGB, step=NGB)
    def _(l):
        for j in range(NGB):
            step(l + j, (NSB + j) % NGB, j % NSB, False, False)

    for l in range(SEQ - NGB, SEQ):
        step(l, l % NGB, l % NSB, False, True)

    for l in range(SEQ - NSB, SEQ):
        wait_scatter(l, l % NSB)


_sc_embed_pe = pl.kernel(
    _sc_body,
    out_type=jax.ShapeDtypeStruct((SEQ, BATCH, D_MODEL), jnp.float32),
    mesh=_MESH,
    scratch_types=_SCRATCH,
)


def kernel(x, table):
    pe = jnp.asarray(_PE)
    xT = x.T
    return _sc_embed_pe(xT, table, pe).transpose(1, 0, 2)
